# baseline (speedup 1.0000x reference)
import functools

import jax
import jax.numpy as jnp
from jax import lax
from jax.experimental import pallas as pl
from jax.experimental.pallas import tpu as pltpu
from jax.experimental.pallas import tpu_sc as plsc

B, N, C = 4, 4096, 2048
L = 16
NW = 32
SEG = (B * C) // NW
NSEG = C // SEG
R = 128
NCHUNK = N // R
JGRP = SEG // L
HALF = JGRP // 2

_mesh = plsc.VectorSubcoreMesh(core_axis_name="c", subcore_axis_name="s")


@functools.partial(
    pl.kernel,
    mesh=_mesh,
    out_type=jax.ShapeDtypeStruct((B, C), jnp.int32),
    scratch_types=[
        pltpu.VMEM((R, SEG), jnp.float32),
        pltpu.VMEM((SEG,), jnp.float32),
        pltpu.VMEM((SEG,), jnp.int32),
    ],
)
def _argmin_sc(x_hbm, out_hbm, buf, minv, mini):
    cid = lax.axis_index("c")
    sid = lax.axis_index("s")
    wid = sid * 2 + cid
    b = wid // NSEG
    c0 = (wid % NSEG) * SEG

    for j in range(JGRP):
        minv[pl.ds(j * L, L)] = jnp.full((L,), jnp.inf, jnp.float32)
        mini[pl.ds(j * L, L)] = jnp.zeros((L,), jnp.int32)

    def chunk_body(g, _):
        pltpu.sync_copy(x_hbm.at[b, pl.ds(g * R, R), pl.ds(c0, SEG)], buf)
        for h in range(2):
            mvs = tuple(minv[pl.ds((h * HALF + jj) * L, L)] for jj in range(HALF))
            mis = tuple(mini[pl.ds((h * HALF + jj) * L, L)] for jj in range(HALF))
            riv0 = jnp.full((L,), g * R, jnp.int32)

            def row_body(r, carry):
                riv, mv, mi = carry
                nmv, nmi = [], []
                for jj in range(HALF):
                    v = buf[r, pl.ds((h * HALF + jj) * L, L)]
                    lt = v < mv[jj]
                    nmv.append(jnp.where(lt, v, mv[jj]))
                    nmi.append(jnp.where(lt, riv, mi[jj]))
                return riv + 1, tuple(nmv), tuple(nmi)

            _, mvs, mis = lax.fori_loop(0, R, row_body, (riv0, mvs, mis))
            for jj in range(HALF):
                minv[pl.ds((h * HALF + jj) * L, L)] = mvs[jj]
                mini[pl.ds((h * HALF + jj) * L, L)] = mis[jj]
        return 0

    lax.fori_loop(0, NCHUNK, chunk_body, 0)
    pltpu.sync_copy(mini, out_hbm.at[b, pl.ds(c0, SEG)])


def kernel(x):
    return _argmin_sc(x).astype(jnp.int64)


# speedup vs baseline: 1.6439x; 1.6439x over previous
import functools

import jax
import jax.numpy as jnp
from jax import lax
from jax.experimental import pallas as pl
from jax.experimental.pallas import tpu as pltpu
from jax.experimental.pallas import tpu_sc as plsc

B, N, C = 4, 4096, 2048
L = 16
NW = 32
SEG = (B * C) // NW
NSEG = C // SEG
R = 128
NCHUNK = N // R
JGRP = SEG // L
HALF = JGRP // 2

_mesh = plsc.VectorSubcoreMesh(core_axis_name="c", subcore_axis_name="s")


@functools.partial(
    pl.kernel,
    mesh=_mesh,
    out_type=jax.ShapeDtypeStruct((B, C), jnp.int32),
    scratch_types=[
        pltpu.VMEM((R, SEG), jnp.float32),
        pltpu.VMEM((R, SEG), jnp.float32),
        pltpu.VMEM((SEG,), jnp.float32),
        pltpu.VMEM((SEG,), jnp.int32),
        pltpu.SemaphoreType.DMA,
        pltpu.SemaphoreType.DMA,
    ],
)
def _argmin_sc(x_hbm, out_hbm, buf0, buf1, minv, mini, sem0, sem1):
    cid = lax.axis_index("c")
    sid = lax.axis_index("s")
    wid = sid * 2 + cid
    b = wid // NSEG
    c0 = (wid % NSEG) * SEG

    def start(g, buf, sem):
        pltpu.async_copy(x_hbm.at[b, pl.ds(g * R, R), pl.ds(c0, SEG)], buf, sem)

    def wait(buf, sem):
        pltpu.make_async_copy(
            x_hbm.at[b, pl.ds(0, R), pl.ds(c0, SEG)], buf, sem
        ).wait()

    for j in range(JGRP):
        minv[pl.ds(j * L, L)] = jnp.full((L,), jnp.inf, jnp.float32)
        mini[pl.ds(j * L, L)] = jnp.zeros((L,), jnp.int32)

    one = jnp.full((L,), 1, jnp.int32)
    two = jnp.full((L,), 2, jnp.int32)

    def compute(g, buf):
        for h in range(2):
            mvs = tuple(minv[pl.ds((h * HALF + jj) * L, L)] for jj in range(HALF))
            mis = tuple(mini[pl.ds((h * HALF + jj) * L, L)] for jj in range(HALF))
            riv0 = jnp.full((L,), g * R, jnp.int32)

            def row_body(r2, carry):
                riv, mv, mi = carry
                rivb = riv + one
                mv, mi = list(mv), list(mi)
                for t in range(2):
                    r = r2 * 2 + t
                    idxv = riv if t == 0 else rivb
                    for jj in range(HALF):
                        v = buf[r, pl.ds((h * HALF + jj) * L, L)]
                        lt = v < mv[jj]
                        mv[jj] = jnp.where(lt, v, mv[jj])
                        mi[jj] = jnp.where(lt, idxv, mi[jj])
                return riv + two, tuple(mv), tuple(mi)

            _, mvs, mis = lax.fori_loop(0, R // 2, row_body, (riv0, mvs, mis))
            for jj in range(HALF):
                minv[pl.ds((h * HALF + jj) * L, L)] = mvs[jj]
                mini[pl.ds((h * HALF + jj) * L, L)] = mis[jj]

    start(0, buf0, sem0)

    def outer(g2, _):
        for t in range(2):
            g = g2 * 2 + t
            bufc, semc = (buf0, sem0) if t == 0 else (buf1, sem1)
            bufn, semn = (buf1, sem1) if t == 0 else (buf0, sem0)

            @pl.when(g + 1 < NCHUNK)
            def _():
                start(g + 1, bufn, semn)

            wait(bufc, semc)
            compute(g, bufc)
        return 0

    lax.fori_loop(0, NCHUNK // 2, outer, 0)
    pltpu.sync_copy(mini, out_hbm.at[b, pl.ds(c0, SEG)])


def kernel(x):
    return _argmin_sc(x).astype(jnp.int64)
